# baseline (speedup 1.0000x reference)
import functools

import jax
import jax.numpy as jnp
from jax import lax
from jax.experimental import pallas as pl
from jax.experimental.pallas import tpu as pltpu
from jax.experimental.pallas import tpu_sc as plsc

_EMB = 32
_K = 5
_J = _K + 1


def _sc_gather(user, item, user_emb, item_emb, user_topk, item_topk):
    B = user.shape[0]
    info = plsc.get_sparse_core_info()
    NC, NS = info.num_cores, info.num_subcores
    NW = NC * NS
    bpw = B // NW
    CH = 128
    NCH = bpw // CH

    mesh = plsc.VectorSubcoreMesh(core_axis_name="c", subcore_axis_name="s")

    @functools.partial(
        pl.kernel,
        out_type=(
            jax.ShapeDtypeStruct((_J, B, _EMB), jnp.float32),
            jax.ShapeDtypeStruct((_J, B, _EMB), jnp.float32),
        ),
        mesh=mesh,
        scratch_types=[
            pltpu.VMEM((bpw,), jnp.int32),
            pltpu.VMEM((2 * CH, _K), jnp.int32),
            pltpu.SemaphoreType.DMA,
            pltpu.SemaphoreType.DMA,
            pltpu.SemaphoreType.DMA,
        ],
    )
    def k(user_h, item_h, uemb_h, iemb_h, utopk_h, itopk_h, out_u, out_i,
          vid_v, tkv_v, sem_tk0, sem_tk1, sem_emb):
        wid = lax.axis_index("s") * NC + lax.axis_index("c")
        base = wid * bpw

        def side(src_h, topk_h, emb_h, out_h):
            pltpu.sync_copy(src_h.at[pl.ds(base, bpw)], vid_v)

            def issue_topk(c, sem):
                par = c % 2
                def grp(g, carry):
                    uvec = vid_v[pl.ds(c * CH + g * 16, 16)]
                    for lane in range(16):
                        tid = uvec[lane]
                        pltpu.async_copy(
                            topk_h.at[pl.ds(tid, 1), :],
                            tkv_v.at[pl.ds(par * CH + g * 16 + lane, 1), :],
                            sem)
                    return carry
                lax.fori_loop(0, CH // 16, grp, 0)

            def drain_topk(sem):
                dummy = pltpu.make_async_copy(
                    topk_h.at[pl.ds(0, 1), :],
                    tkv_v.at[pl.ds(0, 1), :],
                    sem)
                def one(b, carry):
                    dummy.wait()
                    return carry
                lax.fori_loop(0, CH, one, 0, unroll=8)

            sems = (sem_tk0, sem_tk1)
            issue_topk(0, sem_tk0)
            for c in range(NCH):
                par = c % 2
                if c + 1 < NCH:
                    issue_topk(c + 1, sems[(c + 1) % 2])
                drain_topk(sems[par])
                def grp(g, carry, c=c, par=par):
                    uvec = vid_v[pl.ds(c * CH + g * 16, 16)]
                    for lane in range(16):
                        b = g * 16 + lane
                        row = base + c * CH + b
                        tid = uvec[lane]
                        pltpu.async_copy(
                            emb_h.at[pl.ds(tid, 1), :],
                            out_h.at[0, pl.ds(row, 1), :],
                            sem_emb)
                        nvec = tkv_v[par * CH + b, pl.ds(0, 16)]
                        for kk in range(_K):
                            nid = nvec[kk]
                            pltpu.async_copy(
                                emb_h.at[pl.ds(nid, 1), :],
                                out_h.at[kk + 1, pl.ds(row, 1), :],
                                sem_emb)
                    return carry
                lax.fori_loop(0, CH // 16, grp, 0)

        side(user_h, utopk_h, uemb_h, out_u)
        side(item_h, itopk_h, iemb_h, out_i)

        dummy = pltpu.make_async_copy(
            uemb_h.at[pl.ds(0, 1), :],
            out_u.at[0, pl.ds(0, 1), :],
            sem_emb)
        def one(b, carry):
            dummy.wait()
            return carry
        lax.fori_loop(0, 2 * _J * bpw, one, 0, unroll=8)

    return k(user, item, user_emb, item_emb, user_topk, item_topk)


def _tc_mlp(u3, i3, W1u, W1i, b1, W2, b2, W3, b3):
    B = u3.shape[1]
    bB = 2048
    NT = _EMB
    NS2 = _J * _EMB
    H1 = W2.shape[0]
    H2 = W2.shape[1]

    def body(u_ref, i_ref, w1u_ref, w1i_ref,
             b1_ref, w2_ref, b2_ref, w3_ref, b3_ref, o_ref):
        h1 = b1_ref[...]
        for j in range(_J):
            wu = w1u_ref[pl.ds(j * _EMB, _EMB), :]
            wi = w1i_ref[pl.ds(j * _EMB, _EMB), :]
            h1 = h1 + jnp.dot(u_ref[j], wu, preferred_element_type=jnp.float32)
            h1 = h1 + jnp.dot(i_ref[j], wi, preferred_element_type=jnp.float32)
        h1 = jnp.maximum(h1, 0.0)
        h2 = jnp.dot(h1, w2_ref[...], preferred_element_type=jnp.float32)
        h2 = jnp.maximum(h2 + b2_ref[...], 0.0)
        o = jnp.dot(h2, w3_ref[...], preferred_element_type=jnp.float32)
        o_ref[...] = o + b3_ref[0, 0]

    return pl.pallas_call(
        body,
        grid=(B // bB,),
        in_specs=[
            pl.BlockSpec((_J, bB, NT), lambda i: (0, i, 0)),
            pl.BlockSpec((_J, bB, NT), lambda i: (0, i, 0)),
            pl.BlockSpec((NS2, H1), lambda i: (0, 0)),
            pl.BlockSpec((NS2, H1), lambda i: (0, 0)),
            pl.BlockSpec((1, H1), lambda i: (0, 0)),
            pl.BlockSpec((H1, H2), lambda i: (0, 0)),
            pl.BlockSpec((1, H2), lambda i: (0, 0)),
            pl.BlockSpec((H2, 1), lambda i: (0, 0)),
            pl.BlockSpec((1, 1), lambda i: (0, 0)),
        ],
        out_specs=pl.BlockSpec((bB, 1), lambda i: (i, 0)),
        out_shape=jax.ShapeDtypeStruct((B, 1), jnp.float32),
        compiler_params=pltpu.CompilerParams(
            dimension_semantics=("parallel",)),
    )(u3, i3, W1u, W1i, b1, W2, b2, W3, b3)


def kernel(user, item, user_emb, item_emb, user_topk, item_topk,
           W1, b1, W2, b2, W3, b3):
    B = user.shape[0]
    user = user.astype(jnp.int32)
    item = item.astype(jnp.int32)
    user_topk = user_topk.astype(jnp.int32)
    item_topk = item_topk.astype(jnp.int32)

    u3, i3 = _sc_gather(user, item, user_emb, item_emb,
                        user_topk, item_topk)

    NS2 = _J * _EMB
    W1u = W1[:NS2]
    W1i = W1[NS2:]
    out = _tc_mlp(u3, i3, W1u, W1i,
                  b1.reshape(1, -1), W2, b2.reshape(1, -1),
                  W3, b3.reshape(1, 1))
    return out.reshape(B)


# speedup vs baseline: 2.4044x; 2.4044x over previous
import functools

import jax
import jax.numpy as jnp
from jax import lax
from jax.experimental import pallas as pl
from jax.experimental.pallas import tpu as pltpu
from jax.experimental.pallas import tpu_sc as plsc

_EMB = 32
_K = 5
_PAD = 128


def _sc_gather(user, item, user_emb, item_emb, user_topk, item_topk):
    B = user.shape[0]
    info = plsc.get_sparse_core_info()
    NC, NS = info.num_cores, info.num_subcores
    NW = NC * NS
    bpw = B // NW
    npw = bpw * _K

    mesh = plsc.VectorSubcoreMesh(core_axis_name="c", subcore_axis_name="s")

    @functools.partial(
        pl.kernel,
        out_type=(
            jax.ShapeDtypeStruct((B, _PAD), jnp.float32),
            jax.ShapeDtypeStruct((_K, B, _PAD), jnp.float32),
            jax.ShapeDtypeStruct((B, _PAD), jnp.float32),
            jax.ShapeDtypeStruct((_K, B, _PAD), jnp.float32),
        ),
        mesh=mesh,
        scratch_types=[
            pltpu.VMEM((bpw,), jnp.int32),
            pltpu.VMEM((bpw,), jnp.int32),
            pltpu.VMEM((npw,), jnp.int32),
            pltpu.VMEM((npw,), jnp.int32),
            pltpu.VMEM((bpw, _EMB), jnp.float32),
            pltpu.VMEM((npw, _EMB), jnp.float32),
            pltpu.SemaphoreType.DMA,
        ],
        compiler_params=pltpu.CompilerParams(use_tc_tiling_on_sc=False),
    )
    def k(user_h, item_h, uemb_h, iemb_h, utopk_h, itopk_h,
          out_ut, out_un, out_it, out_in,
          uid_v, iid_v, tidx_v, nid_v, targ_v, neib_v, sem):
        wid = lax.axis_index("s") * NC + lax.axis_index("c")
        base = wid * bpw
        pltpu.sync_copy(user_h.at[pl.ds(base, bpw)], uid_v)
        pltpu.sync_copy(item_h.at[pl.ds(base, bpw)], iid_v)

        def side(id_v, topk_h, emb_h, out_t, out_n):
            def build(c, carry):
                idk = id_v[pl.ds(c * 16, 16)] * _K
                for kk in range(_K):
                    tidx_v[pl.ds(kk * bpw + c * 16, 16)] = idk + kk
                return carry

            lax.fori_loop(0, bpw // 16, build, 0)

            pltpu.async_copy(emb_h.at[id_v], targ_v, sem).wait()
            pltpu.sync_copy(targ_v,
                            out_t.at[pl.ds(base, bpw), pl.ds(0, _EMB)])

            pltpu.async_copy(topk_h.at[tidx_v], nid_v, sem).wait()
            pltpu.async_copy(emb_h.at[nid_v], neib_v, sem).wait()
            for kk in range(_K):
                pltpu.sync_copy(
                    neib_v.at[pl.ds(kk * bpw, bpw), :],
                    out_n.at[kk, pl.ds(base, bpw), pl.ds(0, _EMB)])

        side(uid_v, utopk_h, uemb_h, out_ut, out_un)
        side(iid_v, itopk_h, iemb_h, out_it, out_in)

    return k(user, item, user_emb, item_emb,
             user_topk.reshape(-1), item_topk.reshape(-1))


def _tc_mlp(u_t, u_n, i_t, i_n, W1u, W1n_u, W1i, W1n_i, b1, W2, b2, W3, b3):
    B = u_t.shape[0]
    bB = 2048
    NT = _EMB
    NN = _K * _EMB
    H1 = W2.shape[0]
    H2 = W2.shape[1]

    def body(ut_ref, un_ref, it_ref, in_ref,
             w1u_ref, w1un_ref, w1i_ref, w1in_ref,
             b1_ref, w2_ref, b2_ref, w3_ref, b3_ref, o_ref):
        ut = ut_ref[:, :_EMB]
        it = it_ref[:, :_EMB]
        h1 = jnp.dot(ut, w1u_ref[...], preferred_element_type=jnp.float32)
        h1 = h1 + jnp.dot(it, w1i_ref[...], preferred_element_type=jnp.float32)
        for kk in range(_K):
            wu = w1un_ref[pl.ds(kk * _EMB, _EMB), :]
            wi = w1in_ref[pl.ds(kk * _EMB, _EMB), :]
            un = un_ref[kk, :, :_EMB]
            inn = in_ref[kk, :, :_EMB]
            h1 = h1 + jnp.dot(un, wu, preferred_element_type=jnp.float32)
            h1 = h1 + jnp.dot(inn, wi, preferred_element_type=jnp.float32)
        h1 = jnp.maximum(h1 + b1_ref[...], 0.0)
        h2 = jnp.dot(h1, w2_ref[...], preferred_element_type=jnp.float32)
        h2 = jnp.maximum(h2 + b2_ref[...], 0.0)
        o = jnp.dot(h2, w3_ref[...], preferred_element_type=jnp.float32)
        o_ref[...] = o + b3_ref[0, 0]

    nblk = B // bB
    return pl.pallas_call(
        body,
        grid=(nblk,),
        in_specs=[
            pl.BlockSpec((bB, _PAD), lambda i: (i, 0)),
            pl.BlockSpec((_K, bB, _PAD), lambda i: (0, i, 0)),
            pl.BlockSpec((bB, _PAD), lambda i: (i, 0)),
            pl.BlockSpec((_K, bB, _PAD), lambda i: (0, i, 0)),
            pl.BlockSpec((NT, H1), lambda i: (0, 0)),
            pl.BlockSpec((NN, H1), lambda i: (0, 0)),
            pl.BlockSpec((NT, H1), lambda i: (0, 0)),
            pl.BlockSpec((NN, H1), lambda i: (0, 0)),
            pl.BlockSpec((1, H1), lambda i: (0, 0)),
            pl.BlockSpec((H1, H2), lambda i: (0, 0)),
            pl.BlockSpec((1, H2), lambda i: (0, 0)),
            pl.BlockSpec((H2, 1), lambda i: (0, 0)),
            pl.BlockSpec((1, 1), lambda i: (0, 0)),
        ],
        out_specs=pl.BlockSpec((bB, 1), lambda i: (i, 0)),
        out_shape=jax.ShapeDtypeStruct((B, 1), jnp.float32),
        compiler_params=pltpu.CompilerParams(
            dimension_semantics=("parallel",)),
    )(u_t, u_n, i_t, i_n,
      W1u, W1n_u, W1i, W1n_i, b1, W2, b2, W3, b3)


def kernel(user, item, user_emb, item_emb, user_topk, item_topk,
           W1, b1, W2, b2, W3, b3):
    B = user.shape[0]
    user = user.astype(jnp.int32)
    item = item.astype(jnp.int32)
    user_topk = user_topk.astype(jnp.int32)
    item_topk = item_topk.astype(jnp.int32)

    u_t, u_n, i_t, i_n = _sc_gather(
        user, item, user_emb, item_emb, user_topk, item_topk)

    NN = _K * _EMB
    W1u = W1[:_EMB]
    W1n_u = W1[_EMB:_EMB + NN]
    W1i = W1[_EMB + NN:2 * _EMB + NN]
    W1n_i = W1[2 * _EMB + NN:]
    out = _tc_mlp(u_t, u_n, i_t, i_n,
                  W1u, W1n_u, W1i, W1n_i,
                  b1.reshape(1, -1), W2, b2.reshape(1, -1),
                  W3, b3.reshape(1, 1))
    return out.reshape(B)
